# baseline (speedup 1.0000x reference)
import functools

import jax
import jax.numpy as jnp
from jax import lax
from jax.experimental import pallas as pl
from jax.experimental.pallas import tpu as pltpu
from jax.experimental.pallas import tpu_sc as plsc

HIDDEN = 768
LANES = 16
NVEC = HIDDEN // LANES
NC = 2
NS = 16
NW = NC * NS
EPS_EFF = 1e-12 / float(HIDDEN)
INV_H = 1.0 / float(HIDDEN)

CHUNK = 128


def _rsqrt16(v):
    i = lax.bitcast_convert_type(v, jnp.int32)
    i = jnp.int32(0x5F3759DF) - lax.shift_right_arithmetic(i, jnp.int32(1))
    y = lax.bitcast_convert_type(i, jnp.float32)
    for _ in range(3):
        y = y * (1.5 - 0.5 * v * y * y)
    return y


def _lane_sum(x):
    dnums = lax.GatherDimensionNumbers(
        offset_dims=(), collapsed_slice_dims=(0,), start_index_map=(0,))
    for k in (1, 2, 4, 8):
        perm = lax.iota(jnp.int32, 16) ^ k
        xp = lax.gather(x, perm[:, None], dnums, slice_sizes=(1,),
                        mode=lax.GatherScatterMode.PROMISE_IN_BOUNDS)
        x = x + xp
    return x


def _emb_ln_body(ids_h, tab_h, gam_h, bet_h, out_h,
                 idx_v, rows_v, gam_v, bet_v, gsem):
    wid = lax.axis_index("s") * NC + lax.axis_index("c")
    tpw = ids_h.shape[0] // NW
    nchunk = tpw // CHUNK
    base = wid * tpw

    pltpu.sync_copy(gam_h, gam_v)
    pltpu.sync_copy(bet_h, bet_v)

    def chunk_body(c, carry):
        start = base + c * CHUNK
        pltpu.sync_copy(ids_h.at[pl.ds(start, CHUNK)], idx_v)
        pltpu.async_copy(tab_h.at[idx_v], rows_v, gsem).wait()

        def row_body(r, rcarry):
            s = jnp.zeros((LANES,), jnp.float32)
            q = jnp.zeros((LANES,), jnp.float32)
            for j in range(NVEC):
                x = rows_v[r, pl.ds(j * LANES, LANES)]
                s = s + x
                q = q + x * x
            mean_v = _lane_sum(s) * INV_H
            var_v = _lane_sum(q) * INV_H - mean_v * mean_v
            rstd_v = _rsqrt16(var_v + EPS_EFF)
            for j in range(NVEC):
                sl = pl.ds(j * LANES, LANES)
                x = rows_v[r, sl]
                rows_v[r, sl] = (x - mean_v) * rstd_v * gam_v[sl] + bet_v[sl]
            return rcarry

        lax.fori_loop(0, CHUNK, row_body, 0, unroll=False)
        pltpu.sync_copy(rows_v, out_h.at[pl.ds(start, CHUNK)])
        return carry

    lax.fori_loop(0, nchunk, chunk_body, 0, unroll=False)


@functools.partial(jax.jit, static_argnames=())
def _emb_ln(ids, table, gamma, beta):
    n = ids.shape[0]
    mesh = plsc.VectorSubcoreMesh(core_axis_name="c", subcore_axis_name="s")
    fn = pl.kernel(
        _emb_ln_body,
        mesh=mesh,
        out_type=jax.ShapeDtypeStruct((n, HIDDEN), jnp.float32),
        scratch_types=[
            pltpu.VMEM((CHUNK,), jnp.int32),
            pltpu.VMEM((CHUNK, HIDDEN), jnp.float32),
            pltpu.VMEM((HIDDEN,), jnp.float32),
            pltpu.VMEM((HIDDEN,), jnp.float32),
            pltpu.SemaphoreType.DMA,
        ],
    )
    return fn(ids, table, gamma, beta)


def kernel(input_ids, table, gamma, beta):
    ids = input_ids.reshape(-1).astype(jnp.int32)
    out = _emb_ln(ids, table, gamma, beta)
    return out.reshape(input_ids.shape + (HIDDEN,))
